# baseline (speedup 1.0000x reference)
import jax
import jax.numpy as jnp
from jax.experimental import pallas as pl
from jax.experimental.pallas import tpu as pltpu

V, E, B, T, F, H, WIN = 50000, 64, 64, 1024, 128, 128, 3
G3 = 3 * H
HB = B // 2
GC = 64
NCH = T // GC
CH = 32
NC = T // CH


def _embconv_kernel(idx_ref, emb_ref, cwT_ref, cb_ref, feats_ref, xg_ref):
    tc = pl.program_id(0)
    nrows = (GC + 2) * B

    def gbody(g, _):
        row0 = g * 8
        rows = []
        for u in range(8):
            idx = idx_ref[tc, row0 + u]
            rows.append(emb_ref[idx])
        blk = jnp.concatenate(rows, axis=0)
        xg_ref[pl.ds(pl.multiple_of(row0, 8), 8), :] = blk
        return 0

    jax.lax.fori_loop(0, nrows // 8, gbody, 0)

    @pl.when(tc == 0)
    def _():
        xg_ref[0:B, :] = jnp.zeros((B, E), jnp.float32)

    @pl.when(tc == NCH - 1)
    def _():
        xg_ref[(GC + 1) * B:(GC + 2) * B, :] = jnp.zeros((B, E), jnp.float32)

    acc = cb_ref[:].astype(jnp.float32)
    out = acc + sum(
        jnp.dot(xg_ref[w * B:w * B + GC * B, :], cwT_ref[w * E:(w + 1) * E, :],
                preferred_element_type=jnp.float32)
        for w in range(WIN)
    )
    feats_ref[:] = out.reshape(GC, B, F)


def _gru_kernel(featsf_ref, featsb_ref, hid_ref, wihT_ref, whhT_ref,
                bias_ref, bhn_ref, lens_ref, lw_ref, lb_ref, out_ref,
                xpf_ref, xpb_ref, hf_ref, hb_ref, accf_ref, accb_ref):
    c = pl.program_id(0)

    @pl.when(c == 0)
    def _():
        hf_ref[:] = hid_ref[0]
        hb_ref[:] = hid_ref[1]
        accf_ref[:] = jnp.zeros((B, H), jnp.float32)
        accb_ref[:] = jnp.zeros((B, H), jnp.float32)

    xpf = jnp.dot(featsf_ref[:].reshape(CH * B, F), wihT_ref[0],
                  preferred_element_type=jnp.float32) + bias_ref[0]
    xpf_ref[:] = xpf.reshape(CH, B, G3)
    xpb = jnp.dot(featsb_ref[:].reshape(CH * B, F), wihT_ref[1],
                  preferred_element_type=jnp.float32) + bias_ref[1]
    xpb_ref[:] = xpb.reshape(CH, B, G3)

    whhTf = whhT_ref[0]
    whhTb = whhT_ref[1]
    bhnf = bhn_ref[0]
    bhnb = bhn_ref[1]
    lens_v = lens_ref[:]

    def gru_step(xp_j, gh, h, bhn_d, t):
        r = jax.nn.sigmoid(xp_j[:, :H] + gh[:, :H])
        z = jax.nn.sigmoid(xp_j[:, H:2 * H] + gh[:, H:2 * H])
        n = jnp.tanh(xp_j[:, 2 * H:] + r * (gh[:, 2 * H:] + bhn_d))
        hn = (1.0 - z) * n + z * h
        return jnp.where(lens_v > t, hn, h)

    def step(j, carry):
        hf, accf, hb, accb = carry
        sg = c * CH + j
        tf = sg.astype(jnp.float32)
        tb = jnp.float32(T - 1) - tf
        ghf = jnp.dot(hf.astype(jnp.bfloat16), whhTf,
                      preferred_element_type=jnp.float32)
        ghb = jnp.dot(hb.astype(jnp.bfloat16), whhTb,
                      preferred_element_type=jnp.float32)
        hf = gru_step(xpf_ref[j], ghf, hf, bhnf, tf)
        hb = gru_step(xpb_ref[CH - 1 - j], ghb, hb, bhnb, tb)
        return hf, accf + hf, hb, accb + hb

    hf, accf, hb, accb = jax.lax.fori_loop(
        0, CH, step, (hf_ref[:], accf_ref[:], hb_ref[:], accb_ref[:]))
    hf_ref[:] = hf
    hb_ref[:] = hb
    accf_ref[:] = accf
    accb_ref[:] = accb

    @pl.when(c == NC - 1)
    def _():
        lv = lens_v
        rem = jnp.float32(T) - lv
        motf = (accf - rem * hf) / lv
        motb = (accb - rem * hid_ref[1]) / lv
        lw = lw_ref[:]
        col = jnp.sum(motf * lw[0:1, :] + motb * lw[1:2, :],
                      axis=1, keepdims=True)
        out_ref[:] = jax.nn.sigmoid(col + lb_ref[0, 0])


def kernel(ipts, seq_lengths, hidden, emb, conv_w, conv_b,
           w_ih_f, w_hh_f, b_ih_f, b_hh_f,
           w_ih_b, w_hh_b, b_ih_b, b_hh_b, lin_w, lin_b):
    ip_pad = jnp.pad(ipts, ((0, 0), (1, 1)), mode="edge")
    idx_arr = jnp.transpose(
        jnp.stack([ip_pad[:, tc * GC:tc * GC + GC + 2] for tc in range(NCH)]),
        (0, 2, 1)).reshape(NCH, (GC + 2) * B)
    emb3 = emb.reshape(V, 1, E)
    cwT = jnp.transpose(conv_w[:, 0], (1, 2, 0)).reshape(WIN * E, F)
    cb = conv_b.reshape(1, F)

    feats = pl.pallas_call(
        _embconv_kernel,
        grid=(NCH,),
        in_specs=[
            pl.BlockSpec(memory_space=pltpu.SMEM),
            pl.BlockSpec((V, 1, E), lambda tc: (0, 0, 0)),
            pl.BlockSpec((WIN * E, F), lambda tc: (0, 0)),
            pl.BlockSpec((1, F), lambda tc: (0, 0)),
        ],
        out_specs=pl.BlockSpec((GC, B, F), lambda tc: (tc, 0, 0)),
        out_shape=jax.ShapeDtypeStruct((T, B, F), jnp.float32),
        scratch_shapes=[pltpu.VMEM(((GC + 2) * B, E), jnp.float32)],
        compiler_params=pltpu.CompilerParams(
            dimension_semantics=("arbitrary",),
            vmem_limit_bytes=48 * 1024 * 1024,
        ),
    )(idx_arr, emb3, cwT, cb)

    wihT = jnp.stack([w_ih_f.T, w_ih_b.T])
    whhT = jnp.stack([w_hh_f.T, w_hh_b.T]).astype(jnp.bfloat16)
    gate_mask = jnp.arange(G3) < 2 * H
    bias = jnp.stack([
        (b_ih_f + jnp.where(gate_mask, b_hh_f, 0.0)).reshape(1, G3),
        (b_ih_b + jnp.where(gate_mask, b_hh_b, 0.0)).reshape(1, G3),
    ])
    bhn = jnp.stack([b_hh_f[2 * H:].reshape(1, H), b_hh_b[2 * H:].reshape(1, H)])
    lens_f = jnp.broadcast_to(seq_lengths.astype(jnp.float32)[:, None], (B, H))

    out = pl.pallas_call(
        _gru_kernel,
        grid=(NC,),
        in_specs=[
            pl.BlockSpec((CH, B, F), lambda s: (s, 0, 0)),
            pl.BlockSpec((CH, B, F), lambda s: (NC - 1 - s, 0, 0)),
            pl.BlockSpec((2, B, H), lambda s: (0, 0, 0)),
            pl.BlockSpec((2, F, G3), lambda s: (0, 0, 0)),
            pl.BlockSpec((2, H, G3), lambda s: (0, 0, 0)),
            pl.BlockSpec((2, 1, G3), lambda s: (0, 0, 0)),
            pl.BlockSpec((2, 1, H), lambda s: (0, 0, 0)),
            pl.BlockSpec((B, H), lambda s: (0, 0)),
            pl.BlockSpec((2, H), lambda s: (0, 0)),
            pl.BlockSpec((1, 1), lambda s: (0, 0)),
        ],
        out_specs=pl.BlockSpec((B, 1), lambda s: (0, 0)),
        out_shape=jax.ShapeDtypeStruct((B, 1), jnp.float32),
        scratch_shapes=[
            pltpu.VMEM((CH, B, G3), jnp.float32),
            pltpu.VMEM((CH, B, G3), jnp.float32),
            pltpu.VMEM((B, H), jnp.float32),
            pltpu.VMEM((B, H), jnp.float32),
            pltpu.VMEM((B, H), jnp.float32),
            pltpu.VMEM((B, H), jnp.float32),
        ],
        compiler_params=pltpu.CompilerParams(
            dimension_semantics=("arbitrary",),
        ),
    )(feats, feats, hidden, wihT, whhT, bias, bhn, lens_f,
      lin_w.reshape(2, H), lin_b.reshape(1, 1))
    return out.reshape(B)


# speedup vs baseline: 1.0585x; 1.0585x over previous
import jax
import jax.numpy as jnp
from jax.experimental import pallas as pl
from jax.experimental.pallas import tpu as pltpu

V, E, B, T, F, H, WIN = 50000, 64, 64, 1024, 128, 128, 3
G3 = 3 * H
HB = B // 2
GC = 64
NCH = T // GC
CH = 32
NC = T // CH


def _embconv_kernel(idx_ref, emb_ref, cwT_ref, cb_ref, feats_ref, xg_ref):
    tc = pl.program_id(0)
    nrows = (GC + 2) * B

    def gbody(g, _):
        row0 = pl.multiple_of(g * 16, 16)
        for u in range(16):
            idx = idx_ref[tc, g * 16 + u]
            xg_ref[pl.ds(row0 + u, 1), :] = emb_ref[idx]
        return 0

    jax.lax.fori_loop(0, nrows // 16, gbody, 0)

    @pl.when(tc == 0)
    def _():
        xg_ref[0:B, :] = jnp.zeros((B, E), jnp.float32)

    @pl.when(tc == NCH - 1)
    def _():
        xg_ref[(GC + 1) * B:(GC + 2) * B, :] = jnp.zeros((B, E), jnp.float32)

    acc = cb_ref[:].astype(jnp.float32)
    out = acc + sum(
        jnp.dot(xg_ref[w * B:w * B + GC * B, :], cwT_ref[w * E:(w + 1) * E, :],
                preferred_element_type=jnp.float32)
        for w in range(WIN)
    )
    feats_ref[:] = out.reshape(GC, B, F)


def _gru_kernel(featsf_ref, featsb_ref, hid_ref, wihT_ref, whhT_ref,
                bias_ref, bhn_ref, lens_ref, lw_ref, lb_ref, out_ref,
                xpf_ref, xpb_ref, hf_ref, hb_ref, accf_ref, accb_ref):
    c = pl.program_id(0)

    @pl.when(c == 0)
    def _():
        hf_ref[:] = hid_ref[0]
        hb_ref[:] = hid_ref[1]
        accf_ref[:] = jnp.zeros((B, H), jnp.float32)
        accb_ref[:] = jnp.zeros((B, H), jnp.float32)

    xpf = jnp.dot(featsf_ref[:].reshape(CH * B, F), wihT_ref[0],
                  preferred_element_type=jnp.float32) + bias_ref[0]
    xpf_ref[:] = xpf.reshape(CH, B, G3)
    xpb = jnp.dot(featsb_ref[:].reshape(CH * B, F), wihT_ref[1],
                  preferred_element_type=jnp.float32) + bias_ref[1]
    xpb_ref[:] = xpb.reshape(CH, B, G3)

    whhTf = whhT_ref[0]
    whhTb = whhT_ref[1]
    bhnf = bhn_ref[0]
    bhnb = bhn_ref[1]
    lens_v = lens_ref[:]

    def gru_step(xp_j, gh, h, bhn_d, t):
        r = jax.nn.sigmoid(xp_j[:, :H] + gh[:, :H])
        z = jax.nn.sigmoid(xp_j[:, H:2 * H] + gh[:, H:2 * H])
        n = jnp.tanh(xp_j[:, 2 * H:] + r * (gh[:, 2 * H:] + bhn_d))
        hn = (1.0 - z) * n + z * h
        return jnp.where(lens_v > t, hn, h)

    def step(j, carry):
        hf, accf, hb, accb = carry
        sg = c * CH + j
        tf = sg.astype(jnp.float32)
        tb = jnp.float32(T - 1) - tf
        ghf = jnp.dot(hf.astype(jnp.bfloat16), whhTf,
                      preferred_element_type=jnp.float32)
        ghb = jnp.dot(hb.astype(jnp.bfloat16), whhTb,
                      preferred_element_type=jnp.float32)
        hf = gru_step(xpf_ref[j], ghf, hf, bhnf, tf)
        hb = gru_step(xpb_ref[CH - 1 - j], ghb, hb, bhnb, tb)
        return hf, accf + hf, hb, accb + hb

    hf, accf, hb, accb = jax.lax.fori_loop(
        0, CH, step, (hf_ref[:], accf_ref[:], hb_ref[:], accb_ref[:]))
    hf_ref[:] = hf
    hb_ref[:] = hb
    accf_ref[:] = accf
    accb_ref[:] = accb

    @pl.when(c == NC - 1)
    def _():
        lv = lens_v
        rem = jnp.float32(T) - lv
        motf = (accf - rem * hf) / lv
        motb = (accb - rem * hid_ref[1]) / lv
        lw = lw_ref[:]
        col = jnp.sum(motf * lw[0:1, :] + motb * lw[1:2, :],
                      axis=1, keepdims=True)
        out_ref[:] = jax.nn.sigmoid(col + lb_ref[0, 0])


def kernel(ipts, seq_lengths, hidden, emb, conv_w, conv_b,
           w_ih_f, w_hh_f, b_ih_f, b_hh_f,
           w_ih_b, w_hh_b, b_ih_b, b_hh_b, lin_w, lin_b):
    ip_pad = jnp.pad(ipts, ((0, 0), (1, 1)), mode="edge")
    idx_arr = jnp.transpose(
        jnp.stack([ip_pad[:, tc * GC:tc * GC + GC + 2] for tc in range(NCH)]),
        (0, 2, 1)).reshape(NCH, (GC + 2) * B)
    emb3 = emb.reshape(V, 1, E)
    cwT = jnp.transpose(conv_w[:, 0], (1, 2, 0)).reshape(WIN * E, F)
    cb = conv_b.reshape(1, F)

    feats = pl.pallas_call(
        _embconv_kernel,
        grid=(NCH,),
        in_specs=[
            pl.BlockSpec(memory_space=pltpu.SMEM),
            pl.BlockSpec((V, 1, E), lambda tc: (0, 0, 0)),
            pl.BlockSpec((WIN * E, F), lambda tc: (0, 0)),
            pl.BlockSpec((1, F), lambda tc: (0, 0)),
        ],
        out_specs=pl.BlockSpec((GC, B, F), lambda tc: (tc, 0, 0)),
        out_shape=jax.ShapeDtypeStruct((T, B, F), jnp.float32),
        scratch_shapes=[pltpu.VMEM(((GC + 2) * B, E), jnp.float32)],
        compiler_params=pltpu.CompilerParams(
            dimension_semantics=("arbitrary",),
            vmem_limit_bytes=48 * 1024 * 1024,
        ),
    )(idx_arr, emb3, cwT, cb)

    wihT = jnp.stack([w_ih_f.T, w_ih_b.T])
    whhT = jnp.stack([w_hh_f.T, w_hh_b.T]).astype(jnp.bfloat16)
    gate_mask = jnp.arange(G3) < 2 * H
    bias = jnp.stack([
        (b_ih_f + jnp.where(gate_mask, b_hh_f, 0.0)).reshape(1, G3),
        (b_ih_b + jnp.where(gate_mask, b_hh_b, 0.0)).reshape(1, G3),
    ])
    bhn = jnp.stack([b_hh_f[2 * H:].reshape(1, H), b_hh_b[2 * H:].reshape(1, H)])
    lens_f = jnp.broadcast_to(seq_lengths.astype(jnp.float32)[:, None], (B, H))

    out = pl.pallas_call(
        _gru_kernel,
        grid=(NC,),
        in_specs=[
            pl.BlockSpec((CH, B, F), lambda s: (s, 0, 0)),
            pl.BlockSpec((CH, B, F), lambda s: (NC - 1 - s, 0, 0)),
            pl.BlockSpec((2, B, H), lambda s: (0, 0, 0)),
            pl.BlockSpec((2, F, G3), lambda s: (0, 0, 0)),
            pl.BlockSpec((2, H, G3), lambda s: (0, 0, 0)),
            pl.BlockSpec((2, 1, G3), lambda s: (0, 0, 0)),
            pl.BlockSpec((2, 1, H), lambda s: (0, 0, 0)),
            pl.BlockSpec((B, H), lambda s: (0, 0)),
            pl.BlockSpec((2, H), lambda s: (0, 0)),
            pl.BlockSpec((1, 1), lambda s: (0, 0)),
        ],
        out_specs=pl.BlockSpec((B, 1), lambda s: (0, 0)),
        out_shape=jax.ShapeDtypeStruct((B, 1), jnp.float32),
        scratch_shapes=[
            pltpu.VMEM((CH, B, G3), jnp.float32),
            pltpu.VMEM((CH, B, G3), jnp.float32),
            pltpu.VMEM((B, H), jnp.float32),
            pltpu.VMEM((B, H), jnp.float32),
            pltpu.VMEM((B, H), jnp.float32),
            pltpu.VMEM((B, H), jnp.float32),
        ],
        compiler_params=pltpu.CompilerParams(
            dimension_semantics=("arbitrary",),
        ),
    )(feats, feats, hidden, wihT, whhT, bias, bhn, lens_f,
      lin_w.reshape(2, H), lin_b.reshape(1, 1))
    return out.reshape(B)


# speedup vs baseline: 1.0795x; 1.0198x over previous
import jax
import jax.numpy as jnp
from jax.experimental import pallas as pl
from jax.experimental.pallas import tpu as pltpu

V, E, B, T, F, H, WIN = 50000, 64, 64, 1024, 128, 128, 3
G3 = 3 * H
HB = B // 2
GC = 64
NCH = T // GC
CH = 64
NC = T // CH


def _embconv_kernel(idx_ref, emb_ref, cwT_ref, cb_ref, feats_ref, xg_ref):
    tc = pl.program_id(0)
    nrows = (GC + 2) * B

    def gbody(g, _):
        row0 = pl.multiple_of(g * 16, 16)
        for u in range(16):
            idx = idx_ref[tc, g * 16 + u]
            xg_ref[pl.ds(row0 + u, 1), :] = emb_ref[idx]
        return 0

    jax.lax.fori_loop(0, nrows // 16, gbody, 0)

    @pl.when(tc == 0)
    def _():
        xg_ref[0:B, :] = jnp.zeros((B, E), jnp.float32)

    @pl.when(tc == NCH - 1)
    def _():
        xg_ref[(GC + 1) * B:(GC + 2) * B, :] = jnp.zeros((B, E), jnp.float32)

    acc = cb_ref[:].astype(jnp.float32)
    out = acc + sum(
        jnp.dot(xg_ref[w * B:w * B + GC * B, :], cwT_ref[w * E:(w + 1) * E, :],
                preferred_element_type=jnp.float32)
        for w in range(WIN)
    )
    feats_ref[:] = out.reshape(GC, B, F)


def _gru_kernel(featsf_ref, featsb_ref, hid_ref, wihT_ref, whhT_ref,
                bias_ref, bhn_ref, lens_ref, lw_ref, lb_ref, out_ref,
                xpf_ref, xpb_ref, hf_ref, hb_ref, accf_ref, accb_ref):
    c = pl.program_id(0)

    @pl.when(c == 0)
    def _():
        hf_ref[:] = hid_ref[0]
        hb_ref[:] = hid_ref[1]
        accf_ref[:] = jnp.zeros((B, H), jnp.float32)
        accb_ref[:] = jnp.zeros((B, H), jnp.float32)

    xpf = jnp.dot(featsf_ref[:].reshape(CH * B, F), wihT_ref[0],
                  preferred_element_type=jnp.float32) + bias_ref[0]
    xpf_ref[:] = xpf.reshape(CH, B, G3)
    xpb = jnp.dot(featsb_ref[:].reshape(CH * B, F), wihT_ref[1],
                  preferred_element_type=jnp.float32) + bias_ref[1]
    xpb_ref[:] = xpb.reshape(CH, B, G3)

    whhTf = whhT_ref[0]
    whhTb = whhT_ref[1]
    bhnf = bhn_ref[0]
    bhnb = bhn_ref[1]
    lens_v = lens_ref[:]

    def gru_step(xp_j, gh, h, bhn_d, t):
        rz = jnp.tanh((xp_j[:, :2 * H] + gh[:, :2 * H]) * 0.5) * 0.5 + 0.5
        r = rz[:, :H]
        z = rz[:, H:]
        n = jnp.tanh(xp_j[:, 2 * H:] + r * (gh[:, 2 * H:] + bhn_d))
        hn = n + z * (h - n)
        return jnp.where(lens_v > t, hn, h)

    def step(j, carry):
        hf, accf, hb, accb = carry
        sg = c * CH + j
        tf = sg.astype(jnp.float32)
        tb = jnp.float32(T - 1) - tf
        ghf = jnp.dot(hf.astype(jnp.bfloat16), whhTf,
                      preferred_element_type=jnp.float32)
        ghb = jnp.dot(hb.astype(jnp.bfloat16), whhTb,
                      preferred_element_type=jnp.float32)
        hf = gru_step(xpf_ref[j], ghf, hf, bhnf, tf)
        hb = gru_step(xpb_ref[CH - 1 - j], ghb, hb, bhnb, tb)
        return hf, accf + hf, hb, accb + hb

    hf, accf, hb, accb = jax.lax.fori_loop(
        0, CH, step, (hf_ref[:], accf_ref[:], hb_ref[:], accb_ref[:]))
    hf_ref[:] = hf
    hb_ref[:] = hb
    accf_ref[:] = accf
    accb_ref[:] = accb

    @pl.when(c == NC - 1)
    def _():
        lv = lens_v
        rem = jnp.float32(T) - lv
        motf = (accf - rem * hf) / lv
        motb = (accb - rem * hid_ref[1]) / lv
        lw = lw_ref[:]
        col = jnp.sum(motf * lw[0:1, :] + motb * lw[1:2, :],
                      axis=1, keepdims=True)
        out_ref[:] = jax.nn.sigmoid(col + lb_ref[0, 0])


def kernel(ipts, seq_lengths, hidden, emb, conv_w, conv_b,
           w_ih_f, w_hh_f, b_ih_f, b_hh_f,
           w_ih_b, w_hh_b, b_ih_b, b_hh_b, lin_w, lin_b):
    ip_pad = jnp.pad(ipts, ((0, 0), (1, 1)), mode="edge")
    idx_arr = jnp.transpose(
        jnp.stack([ip_pad[:, tc * GC:tc * GC + GC + 2] for tc in range(NCH)]),
        (0, 2, 1)).reshape(NCH, (GC + 2) * B)
    emb3 = emb.reshape(V, 1, E)
    cwT = jnp.transpose(conv_w[:, 0], (1, 2, 0)).reshape(WIN * E, F)
    cb = conv_b.reshape(1, F)

    feats = pl.pallas_call(
        _embconv_kernel,
        grid=(NCH,),
        in_specs=[
            pl.BlockSpec(memory_space=pltpu.SMEM),
            pl.BlockSpec((V, 1, E), lambda tc: (0, 0, 0)),
            pl.BlockSpec((WIN * E, F), lambda tc: (0, 0)),
            pl.BlockSpec((1, F), lambda tc: (0, 0)),
        ],
        out_specs=pl.BlockSpec((GC, B, F), lambda tc: (tc, 0, 0)),
        out_shape=jax.ShapeDtypeStruct((T, B, F), jnp.float32),
        scratch_shapes=[pltpu.VMEM(((GC + 2) * B, E), jnp.float32)],
        compiler_params=pltpu.CompilerParams(
            dimension_semantics=("arbitrary",),
            vmem_limit_bytes=48 * 1024 * 1024,
        ),
    )(idx_arr, emb3, cwT, cb)

    wihT = jnp.stack([w_ih_f.T, w_ih_b.T])
    whhT = jnp.stack([w_hh_f.T, w_hh_b.T]).astype(jnp.bfloat16)
    gate_mask = jnp.arange(G3) < 2 * H
    bias = jnp.stack([
        (b_ih_f + jnp.where(gate_mask, b_hh_f, 0.0)).reshape(1, G3),
        (b_ih_b + jnp.where(gate_mask, b_hh_b, 0.0)).reshape(1, G3),
    ])
    bhn = jnp.stack([b_hh_f[2 * H:].reshape(1, H), b_hh_b[2 * H:].reshape(1, H)])
    lens_f = jnp.broadcast_to(seq_lengths.astype(jnp.float32)[:, None], (B, H))

    out = pl.pallas_call(
        _gru_kernel,
        grid=(NC,),
        in_specs=[
            pl.BlockSpec((CH, B, F), lambda s: (s, 0, 0)),
            pl.BlockSpec((CH, B, F), lambda s: (NC - 1 - s, 0, 0)),
            pl.BlockSpec((2, B, H), lambda s: (0, 0, 0)),
            pl.BlockSpec((2, F, G3), lambda s: (0, 0, 0)),
            pl.BlockSpec((2, H, G3), lambda s: (0, 0, 0)),
            pl.BlockSpec((2, 1, G3), lambda s: (0, 0, 0)),
            pl.BlockSpec((2, 1, H), lambda s: (0, 0, 0)),
            pl.BlockSpec((B, H), lambda s: (0, 0)),
            pl.BlockSpec((2, H), lambda s: (0, 0)),
            pl.BlockSpec((1, 1), lambda s: (0, 0)),
        ],
        out_specs=pl.BlockSpec((B, 1), lambda s: (0, 0)),
        out_shape=jax.ShapeDtypeStruct((B, 1), jnp.float32),
        scratch_shapes=[
            pltpu.VMEM((CH, B, G3), jnp.float32),
            pltpu.VMEM((CH, B, G3), jnp.float32),
            pltpu.VMEM((B, H), jnp.float32),
            pltpu.VMEM((B, H), jnp.float32),
            pltpu.VMEM((B, H), jnp.float32),
            pltpu.VMEM((B, H), jnp.float32),
        ],
        compiler_params=pltpu.CompilerParams(
            dimension_semantics=("arbitrary",),
        ),
    )(feats, feats, hidden, wihT, whhT, bias, bhn, lens_f,
      lin_w.reshape(2, H), lin_b.reshape(1, 1))
    return out.reshape(B)


# speedup vs baseline: 1.1144x; 1.0323x over previous
import jax
import jax.numpy as jnp
from jax.experimental import pallas as pl
from jax.experimental.pallas import tpu as pltpu

V, E, B, T, F, H, WIN = 50000, 64, 64, 1024, 128, 128, 3
G3 = 3 * H
HB = B // 2
GC = 128
NCH = T // GC
CH = 64
NC = T // CH


def _embconv_kernel(idx_ref, emb_ref, cwT_ref, cb_ref, feats_ref, xg_ref):
    tc = pl.program_id(0)
    nrows = (GC + 2) * B

    def gbody(g, _):
        row0 = pl.multiple_of(g * 32, 32)
        for u in range(32):
            idx = idx_ref[tc, g * 32 + u]
            xg_ref[pl.ds(row0 + u, 1), :] = emb_ref[idx]
        return 0

    jax.lax.fori_loop(0, nrows // 32, gbody, 0)

    @pl.when(tc == 0)
    def _():
        xg_ref[0:B, :] = jnp.zeros((B, E), jnp.float32)

    @pl.when(tc == NCH - 1)
    def _():
        xg_ref[(GC + 1) * B:(GC + 2) * B, :] = jnp.zeros((B, E), jnp.float32)

    acc = cb_ref[:].astype(jnp.float32)
    out = acc + sum(
        jnp.dot(xg_ref[w * B:w * B + GC * B, :], cwT_ref[w * E:(w + 1) * E, :],
                preferred_element_type=jnp.float32)
        for w in range(WIN)
    )
    feats_ref[:] = out.reshape(GC, B, F)


def _gru_kernel(featsf_ref, featsb_ref, hid_ref, wihT_ref, whhT_ref,
                bias_ref, bhn_ref, lens_ref, lw_ref, lb_ref, out_ref,
                xpf_ref, xpb_ref, hf_ref, hb_ref, accf_ref, accb_ref):
    c = pl.program_id(0)

    @pl.when(c == 0)
    def _():
        hf_ref[:] = hid_ref[0]
        hb_ref[:] = hid_ref[1]
        accf_ref[:] = jnp.zeros((B, H), jnp.float32)
        accb_ref[:] = jnp.zeros((B, H), jnp.float32)

    xpf = jnp.dot(featsf_ref[:].reshape(CH * B, F), wihT_ref[0],
                  preferred_element_type=jnp.float32) + bias_ref[0]
    xpf_ref[:] = xpf.reshape(CH, B, G3)
    xpb = jnp.dot(featsb_ref[:].reshape(CH * B, F), wihT_ref[1],
                  preferred_element_type=jnp.float32) + bias_ref[1]
    xpb_ref[:] = xpb.reshape(CH, B, G3)

    whhTf = whhT_ref[0]
    whhTb = whhT_ref[1]
    bhnf = bhn_ref[0]
    bhnb = bhn_ref[1]
    lens_v = lens_ref[:]

    def gru_step(xp_j, gh, h, bhn_d, t):
        rz = jnp.tanh((xp_j[:, :2 * H] + gh[:, :2 * H]) * 0.5) * 0.5 + 0.5
        r = rz[:, :H]
        z = rz[:, H:]
        n = jnp.tanh(xp_j[:, 2 * H:] + r * (gh[:, 2 * H:] + bhn_d))
        hn = n + z * (h - n)
        return jnp.where(lens_v > t, hn, h)

    def step(j, carry):
        hf, accf, hb, accb = carry
        sg = c * CH + j
        tf = sg.astype(jnp.float32)
        tb = jnp.float32(T - 1) - tf
        ghf = jnp.dot(hf.astype(jnp.bfloat16), whhTf,
                      preferred_element_type=jnp.float32)
        ghb = jnp.dot(hb.astype(jnp.bfloat16), whhTb,
                      preferred_element_type=jnp.float32)
        hf = gru_step(xpf_ref[j], ghf, hf, bhnf, tf)
        hb = gru_step(xpb_ref[CH - 1 - j], ghb, hb, bhnb, tb)
        return hf, accf + hf, hb, accb + hb

    hf, accf, hb, accb = jax.lax.fori_loop(
        0, CH, step, (hf_ref[:], accf_ref[:], hb_ref[:], accb_ref[:]))
    hf_ref[:] = hf
    hb_ref[:] = hb
    accf_ref[:] = accf
    accb_ref[:] = accb

    @pl.when(c == NC - 1)
    def _():
        lv = lens_v
        rem = jnp.float32(T) - lv
        motf = (accf - rem * hf) / lv
        motb = (accb - rem * hid_ref[1]) / lv
        lw = lw_ref[:]
        col = jnp.sum(motf * lw[0:1, :] + motb * lw[1:2, :],
                      axis=1, keepdims=True)
        out_ref[:] = jax.nn.sigmoid(col + lb_ref[0, 0])


def kernel(ipts, seq_lengths, hidden, emb, conv_w, conv_b,
           w_ih_f, w_hh_f, b_ih_f, b_hh_f,
           w_ih_b, w_hh_b, b_ih_b, b_hh_b, lin_w, lin_b):
    ip_pad = jnp.pad(ipts, ((0, 0), (1, 1)), mode="edge")
    idx_arr = jnp.transpose(
        jnp.stack([ip_pad[:, tc * GC:tc * GC + GC + 2] for tc in range(NCH)]),
        (0, 2, 1)).reshape(NCH, (GC + 2) * B)
    emb3 = emb.reshape(V, 1, E)
    cwT = jnp.transpose(conv_w[:, 0], (1, 2, 0)).reshape(WIN * E, F)
    cb = conv_b.reshape(1, F)

    feats = pl.pallas_call(
        _embconv_kernel,
        grid=(NCH,),
        in_specs=[
            pl.BlockSpec(memory_space=pltpu.SMEM),
            pl.BlockSpec((V, 1, E), lambda tc: (0, 0, 0)),
            pl.BlockSpec((WIN * E, F), lambda tc: (0, 0)),
            pl.BlockSpec((1, F), lambda tc: (0, 0)),
        ],
        out_specs=pl.BlockSpec((GC, B, F), lambda tc: (tc, 0, 0)),
        out_shape=jax.ShapeDtypeStruct((T, B, F), jnp.float32),
        scratch_shapes=[pltpu.VMEM(((GC + 2) * B, E), jnp.float32)],
        compiler_params=pltpu.CompilerParams(
            dimension_semantics=("arbitrary",),
            vmem_limit_bytes=48 * 1024 * 1024,
        ),
    )(idx_arr, emb3, cwT, cb)

    wihT = jnp.stack([w_ih_f.T, w_ih_b.T])
    whhT = jnp.stack([w_hh_f.T, w_hh_b.T]).astype(jnp.bfloat16)
    gate_mask = jnp.arange(G3) < 2 * H
    bias = jnp.stack([
        (b_ih_f + jnp.where(gate_mask, b_hh_f, 0.0)).reshape(1, G3),
        (b_ih_b + jnp.where(gate_mask, b_hh_b, 0.0)).reshape(1, G3),
    ])
    bhn = jnp.stack([b_hh_f[2 * H:].reshape(1, H), b_hh_b[2 * H:].reshape(1, H)])
    lens_f = jnp.broadcast_to(seq_lengths.astype(jnp.float32)[:, None], (B, H))

    out = pl.pallas_call(
        _gru_kernel,
        grid=(NC,),
        in_specs=[
            pl.BlockSpec((CH, B, F), lambda s: (s, 0, 0)),
            pl.BlockSpec((CH, B, F), lambda s: (NC - 1 - s, 0, 0)),
            pl.BlockSpec((2, B, H), lambda s: (0, 0, 0)),
            pl.BlockSpec((2, F, G3), lambda s: (0, 0, 0)),
            pl.BlockSpec((2, H, G3), lambda s: (0, 0, 0)),
            pl.BlockSpec((2, 1, G3), lambda s: (0, 0, 0)),
            pl.BlockSpec((2, 1, H), lambda s: (0, 0, 0)),
            pl.BlockSpec((B, H), lambda s: (0, 0)),
            pl.BlockSpec((2, H), lambda s: (0, 0)),
            pl.BlockSpec((1, 1), lambda s: (0, 0)),
        ],
        out_specs=pl.BlockSpec((B, 1), lambda s: (0, 0)),
        out_shape=jax.ShapeDtypeStruct((B, 1), jnp.float32),
        scratch_shapes=[
            pltpu.VMEM((CH, B, G3), jnp.float32),
            pltpu.VMEM((CH, B, G3), jnp.float32),
            pltpu.VMEM((B, H), jnp.float32),
            pltpu.VMEM((B, H), jnp.float32),
            pltpu.VMEM((B, H), jnp.float32),
            pltpu.VMEM((B, H), jnp.float32),
        ],
        compiler_params=pltpu.CompilerParams(
            dimension_semantics=("arbitrary",),
        ),
    )(feats, feats, hidden, wihT, whhT, bias, bhn, lens_f,
      lin_w.reshape(2, H), lin_b.reshape(1, 1))
    return out.reshape(B)


# speedup vs baseline: 1.1148x; 1.0004x over previous
import jax
import jax.numpy as jnp
from jax.experimental import pallas as pl
from jax.experimental.pallas import tpu as pltpu

V, E, B, T, F, H, WIN = 50000, 64, 64, 1024, 128, 128, 3
G3 = 3 * H
GC = 128
NCH = T // GC
CH = 64
NC = T // CH


def _embconv_kernel(idx_ref, emb_ref, cwT_ref, cb_ref, feats_ref, xg_ref):
    tc = pl.program_id(0)
    nrows = (GC + 2) * B

    def gbody(g, _):
        row0 = pl.multiple_of(g * 32, 32)
        for u in range(32):
            idx = idx_ref[tc, g * 32 + u]
            xg_ref[pl.ds(row0 + u, 1), :] = emb_ref[idx]
        return 0

    jax.lax.fori_loop(0, nrows // 32, gbody, 0)

    @pl.when(tc == 0)
    def _():
        xg_ref[0:B, :] = jnp.zeros((B, E), jnp.float32)

    @pl.when(tc == NCH - 1)
    def _():
        xg_ref[(GC + 1) * B:(GC + 2) * B, :] = jnp.zeros((B, E), jnp.float32)

    acc = cb_ref[:].astype(jnp.float32)
    out = acc + sum(
        jnp.dot(xg_ref[w * B:w * B + GC * B, :], cwT_ref[w * E:(w + 1) * E, :],
                preferred_element_type=jnp.float32)
        for w in range(WIN)
    )
    feats_ref[:] = out.reshape(GC, B, F)


def _gru_kernel(featsf_ref, featsb_ref, hid_ref, wihT_ref, whhT_ref,
                bias_ref, bhn_ref, lens_ref, lw_ref, lb_ref, out_ref,
                xpf_ref, xpb_ref, hf_ref, hb_ref, accf_ref, accb_ref):
    c = pl.program_id(0)

    @pl.when(c == 0)
    def _():
        hf_ref[:] = hid_ref[0]
        hb_ref[:] = hid_ref[1]
        accf_ref[:] = jnp.zeros((B, H), jnp.float32)
        accb_ref[:] = jnp.zeros((B, H), jnp.float32)

    xpf = jnp.dot(featsf_ref[:].reshape(CH * B, F), wihT_ref[0],
                  preferred_element_type=jnp.float32) + bias_ref[0]
    xpf_ref[:] = xpf.reshape(CH, B, G3)
    xpb = jnp.dot(featsb_ref[:].reshape(CH * B, F), wihT_ref[1],
                  preferred_element_type=jnp.float32) + bias_ref[1]
    xpb_ref[:] = xpb.reshape(CH, B, G3)

    whhTf = whhT_ref[0]
    whhTb = whhT_ref[1]
    bhnf = bhn_ref[0]
    bhnb = bhn_ref[1]
    lens_v = lens_ref[:]

    def gru_step(xp_j, gh, h, bhn_d, t):
        rz = jnp.tanh((xp_j[:, :2 * H] + gh[:, :2 * H]) * 0.5) * 0.5 + 0.5
        r = rz[:, :H]
        z = rz[:, H:]
        n = jnp.tanh(xp_j[:, 2 * H:] + r * (gh[:, 2 * H:] + bhn_d))
        hn = n + z * (h - n)
        return jnp.where(lens_v > t, hn, h)

    def step(j, carry):
        hf, accf, hb, accb = carry
        sg = c * CH + j
        tf = sg.astype(jnp.float32)
        tb = jnp.float32(T - 1) - tf
        ghf = jnp.dot(hf.astype(jnp.bfloat16), whhTf,
                      preferred_element_type=jnp.float32)
        ghb = jnp.dot(hb.astype(jnp.bfloat16), whhTb,
                      preferred_element_type=jnp.float32)
        hf = gru_step(xpf_ref[j], ghf, hf, bhnf, tf)
        hb = gru_step(xpb_ref[CH - 1 - j], ghb, hb, bhnb, tb)
        return hf, accf + hf, hb, accb + hb

    hf, accf, hb, accb = jax.lax.fori_loop(
        0, CH, step, (hf_ref[:], accf_ref[:], hb_ref[:], accb_ref[:]))
    hf_ref[:] = hf
    hb_ref[:] = hb
    accf_ref[:] = accf
    accb_ref[:] = accb

    @pl.when(c == NC - 1)
    def _():
        lv = lens_v
        rem = jnp.float32(T) - lv
        motf = (accf - rem * hf) / lv
        motb = (accb - rem * hid_ref[1]) / lv
        lw = lw_ref[:]
        col = jnp.sum(motf * lw[0:1, :] + motb * lw[1:2, :],
                      axis=1, keepdims=True)
        out_ref[:] = jax.nn.sigmoid(col + lb_ref[0, 0])


def kernel(ipts, seq_lengths, hidden, emb, conv_w, conv_b,
           w_ih_f, w_hh_f, b_ih_f, b_hh_f,
           w_ih_b, w_hh_b, b_ih_b, b_hh_b, lin_w, lin_b):
    ip_pad = jnp.pad(ipts, ((0, 0), (1, 1)), mode="edge")
    idx_arr = jnp.transpose(
        jnp.stack([ip_pad[:, tc * GC:tc * GC + GC + 2] for tc in range(NCH)]),
        (0, 2, 1)).reshape(NCH, (GC + 2) * B)
    emb3 = emb.reshape(V, 1, E)
    cwT = jnp.transpose(conv_w[:, 0], (1, 2, 0)).reshape(WIN * E, F)
    cb = conv_b.reshape(1, F)

    feats = pl.pallas_call(
        _embconv_kernel,
        grid=(NCH,),
        in_specs=[
            pl.BlockSpec(memory_space=pltpu.SMEM),
            pl.BlockSpec((V, 1, E), lambda tc: (0, 0, 0)),
            pl.BlockSpec((WIN * E, F), lambda tc: (0, 0)),
            pl.BlockSpec((1, F), lambda tc: (0, 0)),
        ],
        out_specs=pl.BlockSpec((GC, B, F), lambda tc: (tc, 0, 0)),
        out_shape=jax.ShapeDtypeStruct((T, B, F), jnp.float32),
        scratch_shapes=[pltpu.VMEM(((GC + 2) * B, E), jnp.float32)],
        compiler_params=pltpu.CompilerParams(
            dimension_semantics=("arbitrary",),
            vmem_limit_bytes=48 * 1024 * 1024,
        ),
    )(idx_arr, emb3, cwT, cb)

    wihT = jnp.stack([w_ih_f.T, w_ih_b.T])
    whhT = jnp.stack([w_hh_f.T, w_hh_b.T]).astype(jnp.bfloat16)
    gate_mask = jnp.arange(G3) < 2 * H
    bias = jnp.stack([
        (b_ih_f + jnp.where(gate_mask, b_hh_f, 0.0)).reshape(1, G3),
        (b_ih_b + jnp.where(gate_mask, b_hh_b, 0.0)).reshape(1, G3),
    ])
    bhn = jnp.stack([b_hh_f[2 * H:].reshape(1, H), b_hh_b[2 * H:].reshape(1, H)])
    lens_f = jnp.broadcast_to(seq_lengths.astype(jnp.float32)[:, None], (B, H))

    out = pl.pallas_call(
        _gru_kernel,
        grid=(NC,),
        in_specs=[
            pl.BlockSpec((CH, B, F), lambda s: (s, 0, 0)),
            pl.BlockSpec((CH, B, F), lambda s: (NC - 1 - s, 0, 0)),
            pl.BlockSpec((2, B, H), lambda s: (0, 0, 0)),
            pl.BlockSpec((2, F, G3), lambda s: (0, 0, 0)),
            pl.BlockSpec((2, H, G3), lambda s: (0, 0, 0)),
            pl.BlockSpec((2, 1, G3), lambda s: (0, 0, 0)),
            pl.BlockSpec((2, 1, H), lambda s: (0, 0, 0)),
            pl.BlockSpec((B, H), lambda s: (0, 0)),
            pl.BlockSpec((2, H), lambda s: (0, 0)),
            pl.BlockSpec((1, 1), lambda s: (0, 0)),
        ],
        out_specs=pl.BlockSpec((B, 1), lambda s: (0, 0)),
        out_shape=jax.ShapeDtypeStruct((B, 1), jnp.float32),
        scratch_shapes=[
            pltpu.VMEM((CH, B, G3), jnp.float32),
            pltpu.VMEM((CH, B, G3), jnp.float32),
            pltpu.VMEM((B, H), jnp.float32),
            pltpu.VMEM((B, H), jnp.float32),
            pltpu.VMEM((B, H), jnp.float32),
            pltpu.VMEM((B, H), jnp.float32),
        ],
        compiler_params=pltpu.CompilerParams(
            dimension_semantics=("arbitrary",),
        ),
    )(feats, feats, hidden, wihT, whhT, bias, bhn, lens_f,
      lin_w.reshape(2, H), lin_b.reshape(1, 1))
    return out.reshape(B)
